# baseline (speedup 1.0000x reference)
import jax
import jax.numpy as jnp
from jax import lax
from jax.experimental import pallas as pl
from jax.experimental.pallas import tpu as pltpu
from jax.experimental.pallas import tpu_sc as plsc

B = 16384
EMBED = 32
N_GENRES = 8
NC = 2
NS = 16
NW = NC * NS
BPW = B // NW
TITLE_ROW = 100001
T_HALF = B // 2
T_CHUNK = 2048

_MESH_KW = dict(core_axis_name="c", subcore_axis_name="s",
                num_cores=NC, num_subcores=NS)
_PARAMS = dict(use_tc_tiling_on_sc=False, needs_layout_passes=False)


def _genre_body(gtab_hbm, mgT, outG, gidx_v, gtab_v, gacc_v, gi_sem, go_sem):
    wid = lax.axis_index("s") * NC + lax.axis_index("c")
    base = wid * BPW

    gidx_cp = pltpu.async_copy(mgT.at[:, pl.ds(base, BPW)], gidx_v, gi_sem)
    pltpu.sync_copy(gtab_hbm, gtab_v)
    gidx_cp.wait()

    def genre_body(j, _):
        gvs = [gidx_v[g, pl.ds(j * 16, 16)] for g in range(N_GENRES)]
        for d in range(EMBED):
            drow = jnp.full((16,), d, dtype=jnp.int32)
            acc = None
            for g in range(N_GENRES):
                val = plsc.load_gather(gtab_v, [drow, gvs[g]])
                acc = val if acc is None else acc + val
            gacc_v[d, pl.ds(j * 16, 16)] = acc * 0.125
        return _
    lax.fori_loop(0, BPW // 16, genre_body, None)

    pltpu.sync_copy(gacc_v, outG.at[:, pl.ds(base, BPW)])


def _title_body(ttT, mt, outTtl, trow_v, tbuf_v, trow_sem, ti_sem, to_sem):
    wid = lax.axis_index("s") * NC + lax.axis_index("c")

    trow_cp = pltpu.async_copy(ttT.at[pl.ds(wid, 1)], trow_v, trow_sem)
    ti_cp = pltpu.async_copy(mt.at[pl.ds(0, T_HALF)], tbuf_v.at[0], ti_sem)

    trow_cp.wait()
    zrow = jnp.zeros((16,), dtype=jnp.int32)
    for h in range(2):
        hbase = h * T_HALF
        ti_cp.wait()
        for q in range(T_HALF // T_CHUNK):

            def title_body(j, _, q=q):
                for s in range(8):
                    off = q * T_CHUNK + j * 128 + s * 16
                    iv = plsc.bitcast(tbuf_v[0, pl.ds(off, 16)], jnp.int32)
                    tbuf_v[0, pl.ds(off, 16)] = plsc.load_gather(
                        trow_v, [zrow, iv])
                return _
            lax.fori_loop(0, T_CHUNK // 128, title_body, None)
            pltpu.async_copy(
                tbuf_v.at[:, pl.ds(q * T_CHUNK, T_CHUNK)],
                outTtl.at[pl.ds(wid, 1), pl.ds(hbase + q * T_CHUNK, T_CHUNK)],
                to_sem)

        for q in range(T_HALF // T_CHUNK):
            pltpu.make_async_copy(
                tbuf_v.at[:, pl.ds(q * T_CHUNK, T_CHUNK)],
                outTtl.at[pl.ds(wid, 1), pl.ds(hbase + q * T_CHUNK, T_CHUNK)],
                to_sem).wait()
        if h == 0:
            ti_cp = pltpu.async_copy(mt.at[pl.ds(T_HALF, T_HALF)],
                                     tbuf_v.at[0], ti_sem)


@jax.jit
def _run(ttT, gtT, mt_f32, mgT):
    outG = pl.kernel(
        _genre_body,
        out_type=jax.ShapeDtypeStruct((EMBED, B), jnp.float32),
        mesh=plsc.VectorSubcoreMesh(**_MESH_KW),
        scratch_types=[
            pltpu.VMEM((N_GENRES, BPW), jnp.int32),
            pltpu.VMEM((EMBED, 21), jnp.float32),
            pltpu.VMEM((EMBED, BPW), jnp.float32),
            pltpu.SemaphoreType.DMA,
            pltpu.SemaphoreType.DMA,
        ],
        compiler_params=pltpu.CompilerParams(**_PARAMS),
    )(gtT, mgT)
    outTtl = pl.kernel(
        _title_body,
        out_type=jax.ShapeDtypeStruct((EMBED, B), jnp.float32),
        mesh=plsc.VectorSubcoreMesh(**_MESH_KW),
        scratch_types=[
            pltpu.VMEM((1, TITLE_ROW), jnp.float32),
            pltpu.VMEM((1, T_HALF), jnp.float32),
            pltpu.SemaphoreType.DMA,
            pltpu.SemaphoreType.DMA,
            pltpu.SemaphoreType.DMA,
        ],
        compiler_params=pltpu.CompilerParams(**_PARAMS),
    )(ttT, mt_f32)
    return jnp.concatenate([outTtl, outG], axis=0)


def kernel(title_table, genre_table, movie_title, movie_genres):
    mt_f32 = jax.lax.bitcast_convert_type(movie_title.astype(jnp.int32),
                                          jnp.float32)
    outT = _run(title_table.T, genre_table.T, mt_f32,
                movie_genres.astype(jnp.int32).T)
    return outT.T


# speedup vs baseline: 1.0961x; 1.0961x over previous
import jax
import jax.numpy as jnp
from jax import lax
from jax.experimental import pallas as pl
from jax.experimental.pallas import tpu as pltpu
from jax.experimental.pallas import tpu_sc as plsc

B = 16384
EMBED = 32
N_GENRES = 8
NC = 2
NS = 16
NW = NC * NS
BPW = B // NW
TITLE_ROW = 100001
T_HALF = B // 2
T_CHUNK = 2048


def _body(ttT, gtab_flat, mt, mgT, outT,
          trow_v, tbuf_v, gidx_v, gtab_v, gacc_v,
          trow_sem, ti_sem, to_sem, gi_sem, go_sem):
    wid = lax.axis_index("s") * NC + lax.axis_index("c")
    base = wid * BPW

    trow_cp = pltpu.async_copy(ttT.at[pl.ds(wid, 1)], trow_v, trow_sem)
    ti_cp = pltpu.async_copy(mt.at[pl.ds(0, T_HALF)], tbuf_v.at[0], ti_sem)
    gidx_cp = pltpu.async_copy(mgT.at[:, pl.ds(base, BPW)], gidx_v, gi_sem)
    pltpu.sync_copy(gtab_flat, gtab_v)

    gidx_cp.wait()

    def genre_body(j, _):
        gvs = [gidx_v[g, pl.ds(j * 16, 16)] for g in range(N_GENRES)]
        for d in range(EMBED):
            drow = jnp.full((16,), d, dtype=jnp.int32)
            acc = None
            for g in range(N_GENRES):
                val = plsc.load_gather(gtab_v, [drow, gvs[g]])
                acc = val if acc is None else acc + val
            gacc_v[d, pl.ds(j * 16, 16)] = acc * 0.125
        return _
    lax.fori_loop(0, BPW // 16, genre_body, None)

    pltpu.async_copy(gacc_v, outT.at[pl.ds(EMBED, EMBED), pl.ds(base, BPW)],
                     go_sem)

    trow_cp.wait()
    zrow = jnp.zeros((16,), dtype=jnp.int32)
    for h in range(2):
        hbase = h * T_HALF
        ti_cp.wait()
        for q in range(T_HALF // T_CHUNK):

            def title_body(j, _, q=q):
                for s in range(8):
                    off = q * T_CHUNK + j * 128 + s * 16
                    iv = plsc.bitcast(tbuf_v[0, pl.ds(off, 16)], jnp.int32)
                    tbuf_v[0, pl.ds(off, 16)] = plsc.load_gather(
                        trow_v, [zrow, iv])
                return _
            lax.fori_loop(0, T_CHUNK // 128, title_body, None)
            pltpu.async_copy(
                tbuf_v.at[:, pl.ds(q * T_CHUNK, T_CHUNK)],
                outT.at[pl.ds(wid, 1), pl.ds(hbase + q * T_CHUNK, T_CHUNK)],
                to_sem)

        for q in range(T_HALF // T_CHUNK):
            pltpu.make_async_copy(
                tbuf_v.at[:, pl.ds(q * T_CHUNK, T_CHUNK)],
                outT.at[pl.ds(wid, 1), pl.ds(hbase + q * T_CHUNK, T_CHUNK)],
                to_sem).wait()
        if h == 0:
            ti_cp = pltpu.async_copy(mt.at[pl.ds(T_HALF, T_HALF)],
                                     tbuf_v.at[0], ti_sem)

    pltpu.make_async_copy(
        gacc_v, outT.at[pl.ds(EMBED, EMBED), pl.ds(base, BPW)], go_sem).wait()


@jax.jit
def _run(ttT, gtab_flat, mt_f32, mgT):
    mesh = plsc.VectorSubcoreMesh(core_axis_name="c", subcore_axis_name="s",
                                  num_cores=NC, num_subcores=NS)
    return pl.kernel(
        _body,
        out_type=jax.ShapeDtypeStruct((2 * EMBED, B), jnp.float32),
        mesh=mesh,
        scratch_types=[
            pltpu.VMEM((1, TITLE_ROW), jnp.float32),
            pltpu.VMEM((1, T_HALF), jnp.float32),
            pltpu.VMEM((N_GENRES, BPW), jnp.int32),
            pltpu.VMEM((EMBED, 21), jnp.float32),
            pltpu.VMEM((EMBED, BPW), jnp.float32),
            pltpu.SemaphoreType.DMA,
            pltpu.SemaphoreType.DMA,
            pltpu.SemaphoreType.DMA,
            pltpu.SemaphoreType.DMA,
            pltpu.SemaphoreType.DMA,
        ],
        compiler_params=pltpu.CompilerParams(use_tc_tiling_on_sc=False,
                                             needs_layout_passes=False),
    )(ttT, gtab_flat, mt_f32, mgT)


def kernel(title_table, genre_table, movie_title, movie_genres):
    mt_f32 = jax.lax.bitcast_convert_type(movie_title.astype(jnp.int32),
                                          jnp.float32)
    outT = _run(title_table.T, genre_table.T, mt_f32,
                movie_genres.astype(jnp.int32).T)
    return outT.T


# speedup vs baseline: 1.1146x; 1.0169x over previous
import jax
import jax.numpy as jnp
from jax import lax
from jax.experimental import pallas as pl
from jax.experimental.pallas import tpu as pltpu
from jax.experimental.pallas import tpu_sc as plsc

B = 16384
EMBED = 32
N_GENRES = 8
NC = 2
NS = 16
NW = NC * NS
BPW = B // NW
TITLE_ROW = 100001
T_CHUNK = 2048


def _body(ttT, gtab_flat, mt, mgT, outT,
          trow_v, tbuf_v, gidx_v, gtab_v, gacc_v,
          trow_sem, ti_sem, to_sem, gi_sem, go_sem):
    wid = lax.axis_index("s") * NC + lax.axis_index("c")
    base = wid * BPW

    trow_cp = pltpu.async_copy(ttT.at[pl.ds(wid, 1)], trow_v, trow_sem)
    ti_cp = pltpu.async_copy(mt, tbuf_v.at[0], ti_sem)
    gidx_cp = pltpu.async_copy(mgT.at[:, pl.ds(base, BPW)], gidx_v, gi_sem)
    pltpu.sync_copy(gtab_flat, gtab_v)

    gidx_cp.wait()

    def genre_half(h):
        def genre_body(j, _, h=h):
            gvs = [gidx_v[g, pl.ds(h * (BPW // 2) + j * 16, 16)]
                   for g in range(N_GENRES)]
            for d in range(EMBED):
                drow = jnp.full((16,), d, dtype=jnp.int32)
                acc = None
                for g in range(N_GENRES):
                    val = plsc.load_gather(gtab_v, [drow, gvs[g]])
                    acc = val if acc is None else acc + val
                gacc_v[d, pl.ds(j * 16, 16)] = acc * 0.125
            return _
        lax.fori_loop(0, BPW // 32, genre_body, None)

    def gacc_out(h):
        return (gacc_v, outT.at[pl.ds(EMBED, EMBED),
                                pl.ds(base + h * (BPW // 2), BPW // 2)])

    genre_half(0)
    pltpu.async_copy(*gacc_out(0), go_sem)
    pltpu.make_async_copy(*gacc_out(0), go_sem).wait()
    genre_half(1)
    pltpu.async_copy(*gacc_out(1), go_sem)

    trow_cp.wait()
    ti_cp.wait()
    zrow = jnp.zeros((16,), dtype=jnp.int32)
    for q in range(B // T_CHUNK):

        def title_body(j, _, q=q):
            for s in range(8):
                off = q * T_CHUNK + j * 128 + s * 16
                iv = plsc.bitcast(tbuf_v[0, pl.ds(off, 16)], jnp.int32)
                tbuf_v[0, pl.ds(off, 16)] = plsc.load_gather(
                    trow_v, [zrow, iv])
            return _
        lax.fori_loop(0, T_CHUNK // 128, title_body, None)
        pltpu.async_copy(
            tbuf_v.at[:, pl.ds(q * T_CHUNK, T_CHUNK)],
            outT.at[pl.ds(wid, 1), pl.ds(q * T_CHUNK, T_CHUNK)],
            to_sem)

    for q in range(B // T_CHUNK):
        pltpu.make_async_copy(
            tbuf_v.at[:, pl.ds(q * T_CHUNK, T_CHUNK)],
            outT.at[pl.ds(wid, 1), pl.ds(q * T_CHUNK, T_CHUNK)],
            to_sem).wait()
    pltpu.make_async_copy(*gacc_out(1), go_sem).wait()


@jax.jit
def _run(ttT, gtab_flat, mt_f32, mgT):
    mesh = plsc.VectorSubcoreMesh(core_axis_name="c", subcore_axis_name="s",
                                  num_cores=NC, num_subcores=NS)
    return pl.kernel(
        _body,
        out_type=jax.ShapeDtypeStruct((2 * EMBED, B), jnp.float32),
        mesh=mesh,
        scratch_types=[
            pltpu.VMEM((1, TITLE_ROW), jnp.float32),
            pltpu.VMEM((1, B), jnp.float32),
            pltpu.VMEM((N_GENRES, BPW), jnp.int32),
            pltpu.VMEM((EMBED, 21), jnp.float32),
            pltpu.VMEM((EMBED, BPW // 2), jnp.float32),
            pltpu.SemaphoreType.DMA,
            pltpu.SemaphoreType.DMA,
            pltpu.SemaphoreType.DMA,
            pltpu.SemaphoreType.DMA,
            pltpu.SemaphoreType.DMA,
        ],
        compiler_params=pltpu.CompilerParams(use_tc_tiling_on_sc=False,
                                             needs_layout_passes=False),
    )(ttT, gtab_flat, mt_f32, mgT)


def kernel(title_table, genre_table, movie_title, movie_genres):
    mt_f32 = jax.lax.bitcast_convert_type(movie_title.astype(jnp.int32),
                                          jnp.float32)
    outT = _run(title_table.T, genre_table.T, mt_f32,
                movie_genres.astype(jnp.int32).T)
    return outT.T
